# baseline (speedup 1.0000x reference)
import jax
import jax.numpy as jnp
from jax import lax
from jax.experimental import pallas as pl
from jax.experimental.pallas import tpu as pltpu
from jax.experimental.pallas import tpu_sc as plsc

N_ROWS = 65536
DIM = 256
NC = 2
NS = 16
NW = NC * NS
SLAB = N_ROWS // NW
CHUNK = 128
NCHUNK = SLAB // CHUNK
VSEG = SLAB // 16
UNROLL = 8


def _body(states_hbm, mask_hbm, out_hbm, mask_v, idx_v, buf_v, gsem, wsem):
    c = lax.axis_index("c")
    s = lax.axis_index("s")
    wid = s * NC + c
    out_base = wid * SLAB
    iota = lax.iota(jnp.int32, 16)
    zerof = jnp.zeros((16,), jnp.float32)

    pltpu.sync_copy(mask_hbm, mask_v)

    def _count(sg, seg_prefix):
        seg_base = sg * SLAB

        def _sum(i, accs):
            base = seg_base + i * UNROLL * 16
            return tuple(
                accs[u] + mask_v[pl.ds(base + u * 16, 16)]
                for u in range(UNROLL)
            )
        accs = lax.fori_loop(0, VSEG // UNROLL, _sum,
                             (jnp.zeros((16,), jnp.int32),) * UNROLL)
        acc = accs[0]
        for u in range(1, UNROLL):
            acc = acc + accs[u]
        return seg_prefix + jnp.sum(acc)

    total = lax.fori_loop(0, NW, _count, jnp.int32(0))

    q = jnp.clip(total - out_base, 0, SLAB)

    @pl.when(total == N_ROWS)
    def _identity():
        nd = 4
        rows = SLAB // nd
        copies = []
        for k in range(nd):
            off = pl.multiple_of(out_base + k * rows, rows)
            copies.append(pltpu.async_copy(
                states_hbm.at[pl.ds(off, rows)],
                out_hbm.at[pl.ds(off, rows)], gsem))
        for cp in copies:
            cp.wait()

    @pl.when(total < N_ROWS)
    def _general():
        def _zi(i, carry):
            idx_v[pl.ds(i * 16, 16)] = jnp.zeros((16,), jnp.int32)
            return carry
        lax.fori_loop(0, SLAB // 16, _zi, 0)

        def _seg(sg, seg_prefix):
            seg_base = sg * SLAB

            def _sum(i, accs):
                base = seg_base + i * UNROLL * 16
                return tuple(
                    accs[u] + mask_v[pl.ds(base + u * 16, 16)]
                    for u in range(UNROLL)
                )
            accs = lax.fori_loop(0, VSEG // UNROLL, _sum,
                                 (jnp.zeros((16,), jnp.int32),) * UNROLL)
            acc = accs[0]
            for u in range(1, UNROLL):
                acc = acc + accs[u]
            cnt = jnp.sum(acc)

            overlap = ((seg_prefix < out_base + SLAB)
                       & (seg_prefix + cnt > out_base))

            @pl.when(overlap)
            def _compact():
                def _cmp(i, off):
                    m = mask_v[pl.ds(seg_base + i * 16, 16)]
                    mb = m != 0
                    incl = plsc.cumsum(m)
                    pos = seg_prefix + off + incl - m - out_base
                    ids = seg_base + i * 16 + iota
                    keep = mb & (pos >= 0) & (pos < SLAB)
                    plsc.store_scatter(idx_v, [pos], ids, mask=keep)
                    return off + jnp.max(incl)
                lax.fori_loop(0, VSEG, _cmp, jnp.int32(0))

            return seg_prefix + cnt

        lax.fori_loop(0, NW, _seg, jnp.int32(0))

        def _write(cc, src_ref):
            pltpu.async_copy(src_ref, buf_v, gsem).wait()
            dst = pl.multiple_of(out_base + cc * CHUNK, CHUNK)
            pltpu.async_copy(buf_v, out_hbm.at[pl.ds(dst, CHUNK)],
                             wsem).wait()

        nfull = q // CHUNK

        def _cp(cc, carry):
            _write(cc, states_hbm.at[idx_v.at[pl.ds(cc * CHUNK, CHUNK)]])
            return carry
        lax.fori_loop(0, nfull, _cp, 0)

        tail = q - nfull * CHUNK

        @pl.when(tail > 0)
        def _mixed():
            pltpu.async_copy(
                states_hbm.at[idx_v.at[pl.ds(nfull * CHUNK, CHUNK)]],
                buf_v, gsem).wait()

            def _zrow(r, carry):
                for k in range(DIM // 16):
                    buf_v[r, pl.ds(k * 16, 16)] = zerof
                return carry
            lax.fori_loop(tail, CHUNK, _zrow, 0)
            dst = pl.multiple_of(out_base + nfull * CHUNK, CHUNK)
            pltpu.async_copy(
                buf_v, out_hbm.at[pl.ds(dst, CHUNK)], wsem).wait()

        cz0 = nfull + jnp.where(tail > 0, 1, 0)

        @pl.when(cz0 < NCHUNK)
        def _zeros():
            def _zrow(r, carry):
                for k in range(DIM // 16):
                    buf_v[r, pl.ds(k * 16, 16)] = zerof
                return carry
            lax.fori_loop(0, CHUNK, _zrow, 0)

            def _zc(cc, carry):
                dst = pl.multiple_of(out_base + cc * CHUNK, CHUNK)
                pltpu.async_copy(
                    buf_v, out_hbm.at[pl.ds(dst, CHUNK)], wsem).wait()
                return carry
            lax.fori_loop(cz0, NCHUNK, _zc, 0)


_mesh = plsc.VectorSubcoreMesh(core_axis_name="c", subcore_axis_name="s")

_sc_gather = pl.kernel(
    _body,
    out_type=jax.ShapeDtypeStruct((N_ROWS, DIM), jnp.float32),
    mesh=_mesh,
    compiler_params=pltpu.CompilerParams(needs_layout_passes=False),
    scratch_types=[
        pltpu.VMEM((N_ROWS,), jnp.int32),
        pltpu.VMEM((SLAB,), jnp.int32),
        pltpu.VMEM((CHUNK, DIM), jnp.float32),
        pltpu.SemaphoreType.DMA,
        pltpu.SemaphoreType.DMA,
    ],
)


@jax.jit
def kernel(states, active_mask):
    return _sc_gather(states, active_mask.astype(jnp.int32))


# speedup vs baseline: 27.2466x; 27.2466x over previous
import jax
import jax.numpy as jnp
from jax import lax
from jax.experimental import pallas as pl
from jax.experimental.pallas import tpu as pltpu
from jax.experimental.pallas import tpu_sc as plsc

N_ROWS = 65536
DIM = 256
NC = 2
NS = 16
NW = NC * NS
SLAB = N_ROWS // NW
CHUNK = 128
NCHUNK = SLAB // CHUNK
VSEG = SLAB // 16
UNROLL = 8
MBLK = 16384
NBLK = N_ROWS // MBLK


def _body(states_hbm, mask_hbm, out_hbm, mska_v, mskb_v, idx_v,
          buf0_v, buf1_v, gsem, wsem, msem):
    c = lax.axis_index("c")
    s = lax.axis_index("s")
    wid = s * NC + c
    out_base = wid * SLAB
    iota = lax.iota(jnp.int32, 16)
    zerof = jnp.zeros((16,), jnp.float32)

    def _count_block(buf):
        def _sum(i, accs):
            base = i * UNROLL * 16
            return tuple(
                accs[u] + buf[pl.ds(base + u * 16, 16)]
                for u in range(UNROLL)
            )
        accs = lax.fori_loop(0, MBLK // 16 // UNROLL, _sum,
                             (jnp.zeros((16,), jnp.int32),) * UNROLL)
        acc = accs[0]
        for u in range(1, UNROLL):
            acc = acc + accs[u]
        return jnp.sum(acc)

    bufs = (mska_v, mskb_v)
    mc = [None, None]
    mc[0] = pltpu.async_copy(mask_hbm.at[pl.ds(0, MBLK)], mska_v, msem)
    total = jnp.int32(0)
    for blk in range(NBLK):
        mc[blk % 2].wait()
        if blk + 1 < NBLK:
            mc[(blk + 1) % 2] = pltpu.async_copy(
                mask_hbm.at[pl.ds((blk + 1) * MBLK, MBLK)],
                bufs[(blk + 1) % 2], msem)
        total = total + _count_block(bufs[blk % 2])

    q = jnp.clip(total - out_base, 0, SLAB)

    @pl.when(total == N_ROWS)
    def _identity():
        cbuf = (buf0_v, buf1_v)

        def _gather(cc):
            src = pl.multiple_of(out_base + cc * CHUNK, CHUNK)
            return pltpu.async_copy(
                states_hbm.at[pl.ds(src, CHUNK)], cbuf[cc % 2], gsem)

        def _put(cc):
            dst = pl.multiple_of(out_base + cc * CHUNK, CHUNK)
            return pltpu.async_copy(
                cbuf[cc % 2], out_hbm.at[pl.ds(dst, CHUNK)], wsem)

        gc = [None] * NCHUNK
        wc = [None] * NCHUNK
        gc[0] = _gather(0)
        for cc in range(NCHUNK):
            gc[cc].wait()
            if cc + 1 < NCHUNK:
                if cc >= 1:
                    wc[cc - 1].wait()
                gc[cc + 1] = _gather(cc + 1)
            wc[cc] = _put(cc)
        wc[NCHUNK - 2].wait()
        wc[NCHUNK - 1].wait()

    @pl.when(total < N_ROWS)
    def _general():
        def _zi(i, carry):
            idx_v[pl.ds(i * 16, 16)] = jnp.zeros((16,), jnp.int32)
            return carry
        lax.fori_loop(0, SLAB // 16, _zi, 0)

        seg_v = mska_v.at[pl.ds(0, SLAB)]

        def _seg(sg, seg_prefix):
            seg_base = sg * SLAB
            pltpu.sync_copy(mask_hbm.at[pl.ds(seg_base, SLAB)], seg_v)

            def _sum(i, accs):
                base = i * UNROLL * 16
                return tuple(
                    accs[u] + seg_v[pl.ds(base + u * 16, 16)]
                    for u in range(UNROLL)
                )
            accs = lax.fori_loop(0, VSEG // UNROLL, _sum,
                                 (jnp.zeros((16,), jnp.int32),) * UNROLL)
            acc = accs[0]
            for u in range(1, UNROLL):
                acc = acc + accs[u]
            cnt = jnp.sum(acc)

            overlap = ((seg_prefix < out_base + SLAB)
                       & (seg_prefix + cnt > out_base))

            @pl.when(overlap)
            def _compact():
                def _cmp(i, off):
                    m = seg_v[pl.ds(i * 16, 16)]
                    mb = m != 0
                    incl = plsc.cumsum(m)
                    pos = seg_prefix + off + incl - m - out_base
                    ids = seg_base + i * 16 + iota
                    keep = mb & (pos >= 0) & (pos < SLAB)
                    plsc.store_scatter(idx_v, [pos], ids, mask=keep)
                    return off + jnp.max(incl)
                lax.fori_loop(0, VSEG, _cmp, jnp.int32(0))

            return seg_prefix + cnt

        lax.fori_loop(0, NW, _seg, jnp.int32(0))

        def _write(cc, src_ref):
            pltpu.async_copy(src_ref, buf0_v, gsem).wait()
            dst = pl.multiple_of(out_base + cc * CHUNK, CHUNK)
            pltpu.async_copy(buf0_v, out_hbm.at[pl.ds(dst, CHUNK)],
                             wsem).wait()

        nfull = q // CHUNK

        def _cp(cc, carry):
            _write(cc, states_hbm.at[idx_v.at[pl.ds(cc * CHUNK, CHUNK)]])
            return carry
        lax.fori_loop(0, nfull, _cp, 0)

        tail = q - nfull * CHUNK

        @pl.when(tail > 0)
        def _mixed():
            pltpu.async_copy(
                states_hbm.at[idx_v.at[pl.ds(nfull * CHUNK, CHUNK)]],
                buf0_v, gsem).wait()

            def _zrow(r, carry):
                for k in range(DIM // 16):
                    buf0_v[r, pl.ds(k * 16, 16)] = zerof
                return carry
            lax.fori_loop(tail, CHUNK, _zrow, 0)
            dst = pl.multiple_of(out_base + nfull * CHUNK, CHUNK)
            pltpu.async_copy(
                buf0_v, out_hbm.at[pl.ds(dst, CHUNK)], wsem).wait()

        cz0 = nfull + jnp.where(tail > 0, 1, 0)

        @pl.when(cz0 < NCHUNK)
        def _zeros():
            def _zrow(r, carry):
                for k in range(DIM // 16):
                    buf0_v[r, pl.ds(k * 16, 16)] = zerof
                return carry
            lax.fori_loop(0, CHUNK, _zrow, 0)

            def _zc(cc, carry):
                dst = pl.multiple_of(out_base + cc * CHUNK, CHUNK)
                pltpu.async_copy(
                    buf0_v, out_hbm.at[pl.ds(dst, CHUNK)], wsem).wait()
                return carry
            lax.fori_loop(cz0, NCHUNK, _zc, 0)


_mesh = plsc.VectorSubcoreMesh(core_axis_name="c", subcore_axis_name="s")

_sc_gather = pl.kernel(
    _body,
    out_type=jax.ShapeDtypeStruct((N_ROWS, DIM), jnp.float32),
    mesh=_mesh,
    compiler_params=pltpu.CompilerParams(needs_layout_passes=False),
    scratch_types=[
        pltpu.VMEM((MBLK,), jnp.int32),
        pltpu.VMEM((MBLK,), jnp.int32),
        pltpu.VMEM((SLAB,), jnp.int32),
        pltpu.VMEM((CHUNK, DIM), jnp.float32),
        pltpu.VMEM((CHUNK, DIM), jnp.float32),
        pltpu.SemaphoreType.DMA,
        pltpu.SemaphoreType.DMA,
        pltpu.SemaphoreType.DMA,
    ],
)


@jax.jit
def kernel(states, active_mask):
    return _sc_gather(states, active_mask.astype(jnp.int32))
